# baseline (speedup 1.0000x reference)
import functools
import math

import jax
import jax.numpy as jnp
from jax import lax
from jax.experimental import pallas as pl
from jax.experimental.pallas import tpu as pltpu
from jax.experimental.pallas import tpu_sc as plsc

_K = 2
_NUM_EXPERTS = 64
_NUM_BINS = 10
_EXP_BASE = 2.0
_ALIGNMENT = 64
_LANES = 16
_PAD = jnp.iinfo(jnp.int32).max


def _bin_edges(total_tokens):
    start = float(math.ceil(total_tokens / _NUM_EXPERTS))
    stop = float(total_tokens)
    widths = jnp.power(jnp.float32(_EXP_BASE), jnp.arange(0, _NUM_BINS, dtype=jnp.float32))
    normalized = widths / jnp.sum(widths)
    edges = jnp.cumsum(normalized, axis=0)
    edges = start + (stop - start) * edges
    return (jnp.ceil(edges / _ALIGNMENT) * _ALIGNMENT).astype(jnp.int32)


@functools.partial(
    pl.kernel,
    out_type=jax.ShapeDtypeStruct((_LANES,), jnp.int32),
    mesh=plsc.VectorSubcoreMesh(core_axis_name="c", subcore_axis_name="s"),
    scratch_types=[
        pltpu.VMEM((_LANES,), jnp.int32),
        pltpu.VMEM((_LANES,), jnp.int32),
        pltpu.VMEM((_LANES,), jnp.int32),
        pltpu.VMEM((_LANES,), jnp.int32),
    ],
)
def _bucketize_sc(bins_hbm, cap_hbm, last_hbm, out_hbm, bins_v, cap_v, last_v, out_v):
    cid = lax.axis_index("c")
    sid = lax.axis_index("s")

    @pl.when(cid + sid == 0)
    def _():
        pltpu.sync_copy(bins_hbm, bins_v)
        pltpu.sync_copy(cap_hbm, cap_v)
        pltpu.sync_copy(last_hbm, last_v)
        bins = bins_v[...]
        cap = cap_v[...]
        last = last_v[...]
        qual = jnp.where(bins >= cap, bins, last)
        lane = lax.iota(jnp.int32, _LANES)
        dnums = lax.GatherDimensionNumbers(
            offset_dims=(), collapsed_slice_dims=(0,), start_index_map=(0,))
        for step in (1, 2, 4, 8):
            shuf = lax.gather(
                qual, (lane ^ step)[:, None], dnums, slice_sizes=(1,),
                mode=lax.GatherScatterMode.PROMISE_IN_BOUNDS)
            qual = jnp.minimum(qual, shuf)
        out_v[...] = qual
        pltpu.sync_copy(out_v, out_hbm)


def kernel(gate_output, capacity):
    total_tokens = _K * gate_output.shape[0]
    bins = _bin_edges(total_tokens)
    bins16 = jnp.concatenate(
        [bins, jnp.broadcast_to(bins[-1:], (_LANES - _NUM_BINS,))])
    cap16 = jnp.broadcast_to(capacity.astype(jnp.int32), (_LANES,))
    last16 = jnp.broadcast_to(bins[-1:], (_LANES,))
    out16 = _bucketize_sc(bins16, cap16, last16)
    return out16[:1]


# speedup vs baseline: 1.1375x; 1.1375x over previous
import functools
import math

import jax
import jax.numpy as jnp
from jax import lax
from jax.experimental import pallas as pl
from jax.experimental.pallas import tpu as pltpu
from jax.experimental.pallas import tpu_sc as plsc

_K = 2
_NUM_EXPERTS = 64
_NUM_BINS = 10
_EXP_BASE = 2.0
_ALIGNMENT = 64
_LANES = 16

_DNUMS = lax.GatherDimensionNumbers(
    offset_dims=(), collapsed_slice_dims=(0,), start_index_map=(0,))


def _bin_edges(total_tokens):
    start = float(math.ceil(total_tokens / _NUM_EXPERTS))
    stop = float(total_tokens)
    widths = jnp.power(jnp.float32(_EXP_BASE), jnp.arange(0, _NUM_BINS, dtype=jnp.float32))
    normalized = widths / jnp.sum(widths)
    edges = jnp.cumsum(normalized, axis=0)
    edges = start + (stop - start) * edges
    return (jnp.ceil(edges / _ALIGNMENT) * _ALIGNMENT).astype(jnp.int32)


def _shuffle(x, idx):
    return lax.gather(x, idx[:, None], _DNUMS, slice_sizes=(1,),
                      mode=lax.GatherScatterMode.PROMISE_IN_BOUNDS)


@functools.partial(
    pl.kernel,
    out_type=jax.ShapeDtypeStruct((1,), jnp.int32),
    mesh=plsc.VectorSubcoreMesh(core_axis_name="c", subcore_axis_name="s"),
    scratch_types=[
        pltpu.VMEM((_LANES,), jnp.int32),
        pltpu.VMEM((_LANES,), jnp.int32),
        pltpu.VMEM((_LANES,), jnp.int32),
    ],
)
def _bucketize_sc(bins_hbm, cap_hbm, out_hbm, bins_v, cap_v, out_v):
    cid = lax.axis_index("c")
    sid = lax.axis_index("s")

    @pl.when(cid + sid == 0)
    def _():
        pltpu.sync_copy(bins_hbm, bins_v)
        pltpu.sync_copy(cap_hbm, cap_v.at[pl.ds(0, 1)])
        bins = bins_v[...]
        cap = _shuffle(cap_v[...], jnp.zeros((_LANES,), jnp.int32))
        last = _shuffle(bins, jnp.full((_LANES,), _LANES - 1, jnp.int32))
        qual = jnp.where(bins >= cap, bins, last)
        lane = lax.iota(jnp.int32, _LANES)
        for step in (1, 2, 4, 8):
            qual = jnp.minimum(qual, _shuffle(qual, lane ^ step))
        out_v[...] = qual
        pltpu.sync_copy(out_v.at[pl.ds(0, 1)], out_hbm)


def kernel(gate_output, capacity):
    total_tokens = _K * gate_output.shape[0]
    bins = _bin_edges(total_tokens)
    bins16 = jnp.concatenate(
        [bins, jnp.broadcast_to(bins[-1:], (_LANES - _NUM_BINS,))])
    return _bucketize_sc(bins16, capacity.astype(jnp.int32))


# speedup vs baseline: 1.3003x; 1.1431x over previous
import functools
import math

import numpy as np

import jax
import jax.numpy as jnp
from jax import lax
from jax.experimental import pallas as pl
from jax.experimental.pallas import tpu as pltpu
from jax.experimental.pallas import tpu_sc as plsc

_K = 2
_NUM_EXPERTS = 64
_NUM_BINS = 10
_EXP_BASE = 2.0
_ALIGNMENT = 64
_LANES = 16

_DNUMS = lax.GatherDimensionNumbers(
    offset_dims=(), collapsed_slice_dims=(0,), start_index_map=(0,))


def _bin_edges(total_tokens):
    start = np.float32(math.ceil(total_tokens / _NUM_EXPERTS))
    stop = np.float32(total_tokens)
    widths = np.power(np.float32(_EXP_BASE),
                      np.arange(0, _NUM_BINS, dtype=np.float32),
                      dtype=np.float32)
    normalized = (widths / np.sum(widths)).astype(np.float32)
    edges = np.cumsum(normalized, dtype=np.float32)
    edges = (start + (stop - start) * edges).astype(np.float32)
    return (np.ceil(edges / _ALIGNMENT) * _ALIGNMENT).astype(np.int32)


def _shuffle(x, idx):
    return lax.gather(x, idx[:, None], _DNUMS, slice_sizes=(1,),
                      mode=lax.GatherScatterMode.PROMISE_IN_BOUNDS)


def _make_bucketize(bins16):
    mesh = plsc.VectorSubcoreMesh(
        core_axis_name="c", subcore_axis_name="s", num_cores=1)

    @functools.partial(
        pl.kernel,
        out_type=jax.ShapeDtypeStruct((1,), jnp.int32),
        mesh=mesh,
        scratch_types=[
            pltpu.VMEM((_LANES,), jnp.int32),
            pltpu.VMEM((_LANES,), jnp.int32),
        ],
    )
    def _bucketize_sc(cap_hbm, out_hbm, cap_v, out_v):
        cid = lax.axis_index("c")
        sid = lax.axis_index("s")

        @pl.when(cid + sid == 0)
        def _():
            pltpu.sync_copy(cap_hbm, cap_v.at[pl.ds(0, 1)])
            cap = _shuffle(cap_v[...], jnp.zeros((_LANES,), jnp.int32))
            lane = lax.iota(jnp.int32, _LANES)
            last = jnp.full((_LANES,), int(bins16[-1]), jnp.int32)
            bins = last
            for i in range(_NUM_BINS - 1):
                bins = jnp.where(lane == i, int(bins16[i]), bins)
            qual = jnp.where(bins >= cap, bins, last)
            for step in (1, 2, 4, 8):
                qual = jnp.minimum(qual, _shuffle(qual, lane ^ step))
            out_v[...] = qual
            pltpu.sync_copy(out_v.at[pl.ds(0, 1)], out_hbm)

    return _bucketize_sc


def kernel(gate_output, capacity):
    total_tokens = _K * gate_output.shape[0]
    bins = _bin_edges(total_tokens)
    bins16 = np.concatenate(
        [bins, np.broadcast_to(bins[-1:], (_LANES - _NUM_BINS,))])
    return _make_bucketize(bins16)(capacity.astype(jnp.int32))


# speedup vs baseline: 1.4114x; 1.0855x over previous
import functools
import math

import numpy as np

import jax
import jax.numpy as jnp
from jax import lax
from jax.experimental import pallas as pl
from jax.experimental.pallas import tpu as pltpu
from jax.experimental.pallas import tpu_sc as plsc

_K = 2
_NUM_EXPERTS = 64
_NUM_BINS = 10
_EXP_BASE = 2.0
_ALIGNMENT = 64
_LANES = 16

_DNUMS = lax.GatherDimensionNumbers(
    offset_dims=(), collapsed_slice_dims=(0,), start_index_map=(0,))


def _bin_edges(total_tokens):
    start = np.float32(math.ceil(total_tokens / _NUM_EXPERTS))
    stop = np.float32(total_tokens)
    widths = np.power(np.float32(_EXP_BASE),
                      np.arange(0, _NUM_BINS, dtype=np.float32),
                      dtype=np.float32)
    normalized = (widths / np.sum(widths)).astype(np.float32)
    edges = np.cumsum(normalized, dtype=np.float32)
    edges = (start + (stop - start) * edges).astype(np.float32)
    return (np.ceil(edges / _ALIGNMENT) * _ALIGNMENT).astype(np.int32)


def _shuffle(x, idx):
    return lax.gather(x, idx[:, None], _DNUMS, slice_sizes=(1,),
                      mode=lax.GatherScatterMode.PROMISE_IN_BOUNDS)


def _make_bucketize(bins16):
    mesh = plsc.ScalarSubcoreMesh(axis_name="c", num_cores=1)

    @functools.partial(
        pl.kernel,
        out_type=jax.ShapeDtypeStruct((1,), jnp.int32),
        mesh=mesh,
        scratch_types=[
            pltpu.SMEM((1,), jnp.int32),
            pltpu.SMEM((1,), jnp.int32),
        ],
    )
    def _bucketize_sc(cap_hbm, out_hbm, cap_s, out_s):
        pltpu.sync_copy(cap_hbm, cap_s)
        cap = cap_s[0]
        res = jnp.int32(int(bins16[-1]))
        for i in range(_NUM_BINS - 2, -1, -1):
            res = jnp.where(jnp.int32(int(bins16[i])) >= cap,
                            jnp.int32(int(bins16[i])), res)
        out_s[0] = res
        pltpu.sync_copy(out_s, out_hbm)

    return _bucketize_sc


def kernel(gate_output, capacity):
    total_tokens = _K * gate_output.shape[0]
    bins = _bin_edges(total_tokens)
    bins16 = np.concatenate(
        [bins, np.broadcast_to(bins[-1:], (_LANES - _NUM_BINS,))])
    return _make_bucketize(bins16)(capacity.astype(jnp.int32))


# speedup vs baseline: 1.4132x; 1.0013x over previous
import functools
import math

import numpy as np

import jax
import jax.numpy as jnp
from jax import lax
from jax.experimental import pallas as pl
from jax.experimental.pallas import tpu as pltpu
from jax.experimental.pallas import tpu_sc as plsc

_K = 2
_NUM_EXPERTS = 64
_NUM_BINS = 10
_EXP_BASE = 2.0
_ALIGNMENT = 64
_LANES = 16

_DNUMS = lax.GatherDimensionNumbers(
    offset_dims=(), collapsed_slice_dims=(0,), start_index_map=(0,))


def _bin_edges(total_tokens):
    start = np.float32(math.ceil(total_tokens / _NUM_EXPERTS))
    stop = np.float32(total_tokens)
    widths = np.power(np.float32(_EXP_BASE),
                      np.arange(0, _NUM_BINS, dtype=np.float32),
                      dtype=np.float32)
    normalized = (widths / np.sum(widths)).astype(np.float32)
    edges = np.cumsum(normalized, dtype=np.float32)
    edges = (start + (stop - start) * edges).astype(np.float32)
    return (np.ceil(edges / _ALIGNMENT) * _ALIGNMENT).astype(np.int32)


def _shuffle(x, idx):
    return lax.gather(x, idx[:, None], _DNUMS, slice_sizes=(1,),
                      mode=lax.GatherScatterMode.PROMISE_IN_BOUNDS)


def _make_bucketize(bins16):
    mesh = plsc.ScalarSubcoreMesh(axis_name="c", num_cores=1)

    edges = [int(v) for v in bins16[:_NUM_BINS]]

    def _search(cap, lo, hi):
        if lo == hi:
            return jnp.int32(edges[lo])
        mid = (lo + hi) // 2
        return jnp.where(jnp.int32(edges[mid]) >= cap,
                         _search(cap, lo, mid), _search(cap, mid + 1, hi))

    @functools.partial(
        pl.kernel,
        out_type=jax.ShapeDtypeStruct((1,), jnp.int32),
        mesh=mesh,
        scratch_types=[pltpu.SMEM((1,), jnp.int32)],
    )
    def _bucketize_sc(cap_hbm, out_hbm, cap_s):
        pltpu.sync_copy(cap_hbm, cap_s)
        cap_s[0] = _search(cap_s[0], 0, _NUM_BINS - 1)
        pltpu.sync_copy(cap_s, out_hbm)

    return _bucketize_sc


def kernel(gate_output, capacity):
    total_tokens = _K * gate_output.shape[0]
    bins = _bin_edges(total_tokens)
    bins16 = np.concatenate(
        [bins, np.broadcast_to(bins[-1:], (_LANES - _NUM_BINS,))])
    return _make_bucketize(bins16)(capacity.astype(jnp.int32))
